# Initial kernel scaffold; baseline (speedup 1.0000x reference)
import functools

import jax
import jax.numpy as jnp
from jax import lax
from jax.experimental import pallas as pl
from jax.experimental.pallas import tpu as pltpu
from jax.experimental.pallas import tpu_sc as plsc

NC = 2
NS = 16
NW = NC * NS
L = 16
HP = 32
B = 80

f32 = jnp.float32
i32 = jnp.int32


def _mesh():
    return plsc.VectorSubcoreMesh(
        core_axis_name="c", subcore_axis_name="s", num_cores=NC, num_subcores=NS
    )


def _round_up(a, m):
    return (a + m - 1) // m * m


def _sc_deg(dst3, npad, nb):
    nps = npad // NS

    @functools.partial(
        pl.kernel,
        out_type=jax.ShapeDtypeStruct((NC, npad), f32),
        mesh=_mesh(),
        scratch_types=[
            pltpu.VMEM((nb, B), i32),
            pltpu.VMEM((B,), f32),
            pltpu.VMEM((nps,), f32),
            pltpu.VMEM_SHARED((npad,), f32),
        ],
    )
    def deg_kernel(dst_hbm, out_hbm, dstv, ones_v, stage, deg_sh):
        c = lax.axis_index("c")
        s = lax.axis_index("s")
        wid = c * NS + s
        pltpu.sync_copy(dst_hbm.at[wid], dstv)
        one = jnp.ones((L,), f32)
        zero = jnp.zeros((L,), f32)

        def fill_ones(i, carry):
            ones_v[pl.ds(i * L, L)] = one
            return carry

        lax.fori_loop(0, B // L, fill_ones, 0)

        def fill_zero(i, carry):
            stage[pl.ds(i * L, L)] = zero
            return carry

        lax.fori_loop(0, nps // L, fill_zero, 0)
        pltpu.sync_copy(stage, deg_sh.at[pl.ds(s * nps, nps)])
        plsc.subcore_barrier()

        def blk(j, carry):
            pltpu.sync_copy(ones_v, deg_sh.at[dstv.at[j]], add=True)
            return carry

        lax.fori_loop(0, nb, blk, 0)
        plsc.subcore_barrier()
        pltpu.sync_copy(deg_sh.at[pl.ds(s * nps, nps)], stage)
        pltpu.sync_copy(stage, out_hbm.at[c, pl.ds(s * nps, nps)])

    return deg_kernel(dst3)


def _sc_conv(table, src3, dst3, npad, nb):
    nps = npad // NS

    @functools.partial(
        pl.kernel,
        out_type=jax.ShapeDtypeStruct((NC, npad, HP), f32),
        mesh=_mesh(),
        scratch_types=[
            pltpu.VMEM((nb, B), i32),
            pltpu.VMEM((nb, B), i32),
            pltpu.VMEM((B, HP), f32),
            pltpu.VMEM((nps, HP), f32),
            pltpu.VMEM_SHARED((npad, HP), f32),
            pltpu.SemaphoreType.DMA,
        ],
    )
    def conv_kernel(tab_hbm, src_hbm, dst_hbm, out_hbm, srcv, dstv, rows, stage,
                    agg_sh, sem):
        c = lax.axis_index("c")
        s = lax.axis_index("s")
        wid = c * NS + s
        pltpu.sync_copy(src_hbm.at[wid], srcv)
        pltpu.sync_copy(dst_hbm.at[wid], dstv)
        zero = jnp.zeros((L,), f32)

        def fill_zero(rr, carry):
            stage[rr, pl.ds(0, L)] = zero
            stage[rr, pl.ds(L, L)] = zero
            return carry

        lax.fori_loop(0, nps, fill_zero, 0)
        pltpu.sync_copy(stage, agg_sh.at[pl.ds(s * nps, nps)])
        plsc.subcore_barrier()

        def blk(j, carry):
            pltpu.async_copy(tab_hbm.at[srcv.at[j]], rows, sem).wait()
            pltpu.sync_copy(rows, agg_sh.at[dstv.at[j]], add=True)
            return carry

        lax.fori_loop(0, nb, blk, 0)
        plsc.subcore_barrier()
        pltpu.sync_copy(agg_sh.at[pl.ds(s * nps, nps)], stage)
        pltpu.sync_copy(stage, out_hbm.at[c, pl.ds(s * nps, nps)])

    return conv_kernel(table, src3, dst3)


def _sc_hist2(pa, pb, srcf, dst3, npad, nb, epw):
    nps = npad // NS

    @functools.partial(
        pl.kernel,
        out_type=jax.ShapeDtypeStruct((NC, 2, npad), f32),
        mesh=_mesh(),
        scratch_types=[
            pltpu.VMEM((npad,), f32),
            pltpu.VMEM((npad,), f32),
            pltpu.VMEM((epw,), i32),
            pltpu.VMEM((nb, B), i32),
            pltpu.VMEM((epw,), f32),
            pltpu.VMEM((epw,), f32),
            pltpu.VMEM((nps,), f32),
            pltpu.VMEM_SHARED((npad,), f32),
            pltpu.VMEM_SHARED((npad,), f32),
        ],
    )
    def hist_kernel(pa_hbm, pb_hbm, src_hbm, dst_hbm, out_hbm, pa_t, pb_t,
                    srcv, dstv, va, vb, stage, sh_a, sh_b):
        c = lax.axis_index("c")
        s = lax.axis_index("s")
        wid = c * NS + s
        pltpu.sync_copy(pa_hbm, pa_t)
        pltpu.sync_copy(pb_hbm, pb_t)
        pltpu.sync_copy(src_hbm.at[wid], srcv)
        pltpu.sync_copy(dst_hbm.at[wid], dstv)
        zero = jnp.zeros((L,), f32)

        def fill_zero(i, carry):
            stage[pl.ds(i * L, L)] = zero
            return carry

        lax.fori_loop(0, nps // L, fill_zero, 0)
        pltpu.sync_copy(stage, sh_a.at[pl.ds(s * nps, nps)])
        pltpu.sync_copy(stage, sh_b.at[pl.ds(s * nps, nps)])
        plsc.subcore_barrier()

        def gat(i, carry):
            sl = pl.ds(i * L, L)
            idx = srcv[sl]
            va[sl] = plsc.load_gather(pa_t, [idx])
            vb[sl] = plsc.load_gather(pb_t, [idx])
            return carry

        lax.fori_loop(0, epw // L, gat, 0)

        def blk(j, carry):
            sl = pl.ds(j * B, B)
            pltpu.sync_copy(va.at[sl], sh_a.at[dstv.at[j]], add=True)
            pltpu.sync_copy(vb.at[sl], sh_b.at[dstv.at[j]], add=True)
            return carry

        lax.fori_loop(0, nb, blk, 0)
        plsc.subcore_barrier()
        sl = pl.ds(s * nps, nps)
        pltpu.sync_copy(sh_a.at[sl], stage)
        pltpu.sync_copy(stage, out_hbm.at[c, 0, sl])
        pltpu.sync_copy(sh_b.at[sl], stage)
        pltpu.sync_copy(stage, out_hbm.at[c, 1, sl])

    return hist_kernel(pa, pb, srcf, dst3)


def _sc_final(P, pa, pb, dinv, srcf, dstf, cflat, npad, epw):
    @functools.partial(
        pl.kernel,
        out_type=jax.ShapeDtypeStruct((NW, epw), f32),
        mesh=_mesh(),
        scratch_types=[
            pltpu.VMEM((npad,), f32),
            pltpu.VMEM((npad,), f32),
            pltpu.VMEM((npad,), f32),
            pltpu.VMEM((npad,), f32),
            pltpu.VMEM((npad,), f32),
            pltpu.VMEM((epw,), i32),
            pltpu.VMEM((epw,), i32),
            pltpu.VMEM((epw,), f32),
            pltpu.VMEM((epw,), f32),
        ],
    )
    def final_kernel(p_hbm, pa_hbm, pb_hbm, dinv_hbm, src_hbm, dst_hbm, c_hbm,
                     out_hbm, sa, sb, t0, t1, dv, srcv, dstv, cv, ov):
        c = lax.axis_index("c")
        s = lax.axis_index("s")
        wid = c * NS + s
        pltpu.sync_copy(dinv_hbm, dv)
        pltpu.sync_copy(p_hbm.at[0, 0], sa)
        pltpu.sync_copy(p_hbm.at[1, 0], t0)
        pltpu.sync_copy(pa_hbm, t1)

        def ca(i, carry):
            sl = pl.ds(i * L, L)
            sa[sl] = (sa[sl] + t0[sl] + t1[sl]) * dv[sl]
            return carry

        lax.fori_loop(0, npad // L, ca, 0)
        pltpu.sync_copy(p_hbm.at[0, 1], sb)
        pltpu.sync_copy(p_hbm.at[1, 1], t0)
        pltpu.sync_copy(pb_hbm, t1)

        def cb(i, carry):
            sl = pl.ds(i * L, L)
            sb[sl] = (sb[sl] + t0[sl] + t1[sl]) * dv[sl]
            return carry

        lax.fori_loop(0, npad // L, cb, 0)
        pltpu.sync_copy(src_hbm.at[wid], srcv)
        pltpu.sync_copy(dst_hbm.at[wid], dstv)
        pltpu.sync_copy(c_hbm.at[wid], cv)

        def gat(i, carry):
            sl = pl.ds(i * L, L)
            ia = srcv[sl]
            ib = dstv[sl]
            ov[sl] = (plsc.load_gather(sa, [ia]) + plsc.load_gather(sb, [ib])
                      + cv[sl])
            return carry

        lax.fori_loop(0, epw // L, gat, 0)
        pltpu.sync_copy(ov, out_hbm.at[wid])

    return final_kernel(P, pa, pb, dinv, srcf, dstf, cflat)


def _tc_edge(ea, We1p, be1p, wcp, consts, be):
    epad, de = ea.shape
    nblk = epad // be

    def body(ea_ref, w_ref, b_ref, wc_ref, k_ref, out_ref):
        t = jnp.dot(ea_ref[...], w_ref[...], preferred_element_type=f32)
        t = jnp.maximum(t + b_ref[...], 0.0)
        out_ref[0, :] = jnp.sum(t * wc_ref[...], axis=1) + k_ref[0, 0]

    return pl.pallas_call(
        body,
        grid=(nblk,),
        in_specs=[
            pl.BlockSpec((be, de), lambda i: (i, 0)),
            pl.BlockSpec((de, HP), lambda i: (0, 0)),
            pl.BlockSpec((1, HP), lambda i: (0, 0)),
            pl.BlockSpec((1, HP), lambda i: (0, 0)),
            pl.BlockSpec(memory_space=pltpu.SMEM),
        ],
        out_specs=pl.BlockSpec((1, be), lambda i: (i, 0)),
        out_shape=jax.ShapeDtypeStruct((nblk, be), f32),
    )(ea, We1p, be1p, wcp, consts)


def _tc_node1(xp, W1p, deg2, npad, r):
    nblk = npad // r
    d = xp.shape[1]

    def body(x_ref, w_ref, deg_ref, xs_ref, dinv_ref):
        deg = deg_ref[0, :] + deg_ref[1, :] + 1.0
        dinv = lax.rsqrt(deg)
        xw = jnp.dot(x_ref[...], w_ref[...], preferred_element_type=f32)
        xs_ref[...] = xw * dinv[:, None]
        dinv_ref[0, :] = dinv

    return pl.pallas_call(
        body,
        grid=(nblk,),
        in_specs=[
            pl.BlockSpec((r, d), lambda i: (i, 0)),
            pl.BlockSpec((d, HP), lambda i: (0, 0)),
            pl.BlockSpec((2, r), lambda i: (0, i)),
        ],
        out_specs=[
            pl.BlockSpec((r, HP), lambda i: (i, 0)),
            pl.BlockSpec((1, r), lambda i: (i, 0)),
        ],
        out_shape=[
            jax.ShapeDtypeStruct((npad, HP), f32),
            jax.ShapeDtypeStruct((nblk, r), f32),
        ],
    )(xp, W1p, deg2)


def _tc_node2(agg, xs1, dinv2, b1p, w2ab, npad, r):
    nblk = npad // r

    def body(agg_ref, xs_ref, dinv_ref, b_ref, w_ref, pa_ref, pb_ref):
        dinv = dinv_ref[0, :]
        pre = (agg_ref[0] + agg_ref[1] + xs_ref[...]) * dinv[:, None]
        h1 = jnp.maximum(pre + b_ref[...], 0.0)
        pa_ref[0, :] = dinv * jnp.sum(h1 * w_ref[0:1, :], axis=1)
        pb_ref[0, :] = dinv * jnp.sum(h1 * w_ref[1:2, :], axis=1)

    return pl.pallas_call(
        body,
        grid=(nblk,),
        in_specs=[
            pl.BlockSpec((2, r, HP), lambda i: (0, i, 0)),
            pl.BlockSpec((r, HP), lambda i: (i, 0)),
            pl.BlockSpec((1, r), lambda i: (i, 0)),
            pl.BlockSpec((1, HP), lambda i: (0, 0)),
            pl.BlockSpec((2, HP), lambda i: (0, 0)),
        ],
        out_specs=[
            pl.BlockSpec((1, r), lambda i: (i, 0)),
            pl.BlockSpec((1, r), lambda i: (i, 0)),
        ],
        out_shape=[
            jax.ShapeDtypeStruct((nblk, r), f32),
            jax.ShapeDtypeStruct((nblk, r), f32),
        ],
    )(agg, xs1, dinv2, b1p, w2ab)


def kernel(x, edge_index, edge_attr, W1, b1, W2, b2, We1, be1, We2, be2, Wf, bf):
    n, d = x.shape
    e = edge_index.shape[1]
    h = W1.shape[1]

    r = 1024
    npad = _round_up(n + 1, r)
    nb = _round_up(e, NW * B) // (NW * B)
    epw = nb * B
    epad = NW * epw
    be = max(NW * B, 2560)
    while epad % be:
        be //= 2

    pad_h = HP - h
    W1p = jnp.pad(W1, ((0, 0), (0, pad_h)))
    We1p = jnp.pad(We1, ((0, 0), (0, pad_h)))
    be1p = jnp.pad(be1, (0, pad_h)).reshape(1, HP)
    b1p = jnp.pad(b1, (0, pad_h)).reshape(1, HP)
    Wfa, Wfb, Wfc = Wf[:h, 0], Wf[h:2 * h, 0], Wf[2 * h:, 0]
    w2ab = jnp.pad(jnp.stack([W2 @ Wfa, W2 @ Wfb]), ((0, 0), (0, pad_h)))
    wcp = jnp.pad(We2 @ Wfc, (0, pad_h)).reshape(1, HP)
    consts = (be2 @ Wfc + bf[0] + b2 @ Wfa + b2 @ Wfb).reshape(1, 1)

    xp = jnp.pad(x, ((0, npad - n), (0, 0)))
    eap = jnp.pad(edge_attr, ((0, epad - e), (0, 0)))
    src = jnp.pad(edge_index[0], (0, epad - e), constant_values=n)
    dst = jnp.pad(edge_index[1], (0, epad - e), constant_values=n)
    src3 = src.reshape(NW, nb, B)
    dst3 = dst.reshape(NW, nb, B)
    srcf = src.reshape(NW, epw)
    dstf = dst.reshape(NW, epw)

    deg2 = _sc_deg(dst3, npad, nb)
    cflat = _tc_edge(eap, We1p, be1p, wcp, consts, be).reshape(NW, epw)
    xs1, dinv2 = _tc_node1(xp, W1p, deg2, npad, r)
    agg = _sc_conv(xs1, src3, dst3, npad, nb)
    pa2, pb2 = _tc_node2(agg, xs1, dinv2, b1p, w2ab, npad, r)
    P = _sc_hist2(pa2.reshape(npad), pb2.reshape(npad), srcf, dst3, npad, nb,
                  epw)
    pred = _sc_final(P, pa2.reshape(npad), pb2.reshape(npad),
                     dinv2.reshape(npad), srcf, dstf, cflat, npad, epw)
    return pred.reshape(epad)[:e]


# speedup vs baseline: 21.9097x; 21.9097x over previous
import functools

import jax
import jax.numpy as jnp
from jax import lax
from jax.experimental import pallas as pl
from jax.experimental.pallas import tpu as pltpu
from jax.experimental.pallas import tpu_sc as plsc

NC = 2
NS = 16
NW = NC * NS
L = 16
HP = 32
B = 80

f32 = jnp.float32
i32 = jnp.int32


def _mesh():
    return plsc.VectorSubcoreMesh(
        core_axis_name="c", subcore_axis_name="s", num_cores=NC, num_subcores=NS
    )


def _round_up(a, m):
    return (a + m - 1) // m * m


def _sc_deg(dst3, npad, nb):
    nps = npad // NS

    @functools.partial(
        pl.kernel,
        out_type=jax.ShapeDtypeStruct((NC, npad), f32),
        mesh=_mesh(),
        scratch_types=[
            pltpu.VMEM((nb, B), i32),
            pltpu.VMEM((B,), f32),
            pltpu.VMEM((nps,), f32),
            pltpu.VMEM_SHARED((npad,), f32),
        ],
    )
    def deg_kernel(dst_hbm, out_hbm, dstv, ones_v, stage, deg_sh):
        c = lax.axis_index("c")
        s = lax.axis_index("s")
        wid = c * NS + s
        pltpu.sync_copy(dst_hbm.at[wid], dstv)
        one = jnp.ones((L,), f32)
        zero = jnp.zeros((L,), f32)

        def fill_ones(i, carry):
            ones_v[pl.ds(i * L, L)] = one
            return carry

        lax.fori_loop(0, B // L, fill_ones, 0)

        def fill_zero(i, carry):
            stage[pl.ds(i * L, L)] = zero
            return carry

        lax.fori_loop(0, nps // L, fill_zero, 0)
        pltpu.sync_copy(stage, deg_sh.at[pl.ds(s * nps, nps)])
        plsc.subcore_barrier()

        def blk(j, carry):
            pltpu.sync_copy(ones_v, deg_sh.at[dstv.at[j]], add=True)
            return carry

        lax.fori_loop(0, nb, blk, 0)
        plsc.subcore_barrier()
        pltpu.sync_copy(deg_sh.at[pl.ds(s * nps, nps)], stage)
        pltpu.sync_copy(stage, out_hbm.at[c, pl.ds(s * nps, nps)])

    return deg_kernel(dst3)


def _sc_conv(table, src3, dst3, npad, nb):
    nps = npad // NS

    @functools.partial(
        pl.kernel,
        out_type=jax.ShapeDtypeStruct((NC, npad, HP), f32),
        mesh=_mesh(),
        scratch_types=[
            pltpu.VMEM((nb, B), i32),
            pltpu.VMEM((nb, B), i32),
            pltpu.VMEM((B, HP), f32),
            pltpu.VMEM((nps, HP), f32),
            pltpu.VMEM_SHARED((npad, HP), f32),
            pltpu.SemaphoreType.DMA,
        ],
        compiler_params=pltpu.CompilerParams(use_tc_tiling_on_sc=False),
    )
    def conv_kernel(tab_hbm, src_hbm, dst_hbm, out_hbm, srcv, dstv, rows, stage,
                    agg_sh, sem):
        c = lax.axis_index("c")
        s = lax.axis_index("s")
        wid = c * NS + s
        pltpu.sync_copy(src_hbm.at[wid], srcv)
        pltpu.sync_copy(dst_hbm.at[wid], dstv)
        zero = jnp.zeros((L,), f32)

        def fill_zero(rr, carry):
            stage[rr, pl.ds(0, L)] = zero
            stage[rr, pl.ds(L, L)] = zero
            return carry

        lax.fori_loop(0, nps, fill_zero, 0)
        pltpu.sync_copy(stage, agg_sh.at[pl.ds(s * nps, nps)])
        plsc.subcore_barrier()

        def blk(j, carry):
            pltpu.async_copy(tab_hbm.at[srcv.at[j]], rows, sem).wait()
            pltpu.sync_copy(rows, agg_sh.at[dstv.at[j]], add=True)
            return carry

        lax.fori_loop(0, nb, blk, 0)
        plsc.subcore_barrier()
        pltpu.sync_copy(agg_sh.at[pl.ds(s * nps, nps)], stage)
        pltpu.sync_copy(stage, out_hbm.at[c, pl.ds(s * nps, nps)])

    return conv_kernel(table, src3, dst3)


def _sc_hist2(pa, pb, srcf, dst3, npad, nb, epw):
    nps = npad // NS

    @functools.partial(
        pl.kernel,
        out_type=jax.ShapeDtypeStruct((NC, 2, npad), f32),
        mesh=_mesh(),
        scratch_types=[
            pltpu.VMEM((npad,), f32),
            pltpu.VMEM((npad,), f32),
            pltpu.VMEM((epw,), i32),
            pltpu.VMEM((nb, B), i32),
            pltpu.VMEM((epw,), f32),
            pltpu.VMEM((epw,), f32),
            pltpu.VMEM((nps,), f32),
            pltpu.VMEM_SHARED((npad,), f32),
            pltpu.VMEM_SHARED((npad,), f32),
        ],
        compiler_params=pltpu.CompilerParams(needs_layout_passes=False),
    )
    def hist_kernel(pa_hbm, pb_hbm, src_hbm, dst_hbm, out_hbm, pa_t, pb_t,
                    srcv, dstv, va, vb, stage, sh_a, sh_b):
        c = lax.axis_index("c")
        s = lax.axis_index("s")
        wid = c * NS + s
        pltpu.sync_copy(pa_hbm, pa_t)
        pltpu.sync_copy(pb_hbm, pb_t)
        pltpu.sync_copy(src_hbm.at[wid], srcv)
        pltpu.sync_copy(dst_hbm.at[wid], dstv)
        zero = jnp.zeros((L,), f32)

        def fill_zero(i, carry):
            stage[pl.ds(i * L, L)] = zero
            return carry

        lax.fori_loop(0, nps // L, fill_zero, 0)
        pltpu.sync_copy(stage, sh_a.at[pl.ds(s * nps, nps)])
        pltpu.sync_copy(stage, sh_b.at[pl.ds(s * nps, nps)])
        plsc.subcore_barrier()

        def gat(i, carry):
            sl = pl.ds(i * L, L)
            idx = srcv[sl]
            va[sl] = plsc.load_gather(pa_t, [idx])
            vb[sl] = plsc.load_gather(pb_t, [idx])
            return carry

        lax.fori_loop(0, epw // L, gat, 0)

        def blk(j, carry):
            sl = pl.ds(j * B, B)
            pltpu.sync_copy(va.at[sl], sh_a.at[dstv.at[j]], add=True)
            pltpu.sync_copy(vb.at[sl], sh_b.at[dstv.at[j]], add=True)
            return carry

        lax.fori_loop(0, nb, blk, 0)
        plsc.subcore_barrier()
        sl = pl.ds(s * nps, nps)
        pltpu.sync_copy(sh_a.at[sl], stage)
        pltpu.sync_copy(stage, out_hbm.at[c, 0, sl])
        pltpu.sync_copy(sh_b.at[sl], stage)
        pltpu.sync_copy(stage, out_hbm.at[c, 1, sl])

    return hist_kernel(pa, pb, srcf, dst3)


def _sc_final(P, pa, pb, dinv, srcf, dstf, cflat, npad, epw):
    @functools.partial(
        pl.kernel,
        out_type=jax.ShapeDtypeStruct((NW, epw), f32),
        mesh=_mesh(),
        scratch_types=[
            pltpu.VMEM((npad,), f32),
            pltpu.VMEM((npad,), f32),
            pltpu.VMEM((npad,), f32),
            pltpu.VMEM((npad,), f32),
            pltpu.VMEM((npad,), f32),
            pltpu.VMEM((epw,), i32),
            pltpu.VMEM((epw,), i32),
            pltpu.VMEM((epw,), f32),
            pltpu.VMEM((epw,), f32),
        ],
        compiler_params=pltpu.CompilerParams(needs_layout_passes=False),
    )
    def final_kernel(p_hbm, pa_hbm, pb_hbm, dinv_hbm, src_hbm, dst_hbm, c_hbm,
                     out_hbm, sa, sb, t0, t1, dv, srcv, dstv, cv, ov):
        c = lax.axis_index("c")
        s = lax.axis_index("s")
        wid = c * NS + s
        pltpu.sync_copy(dinv_hbm, dv)
        pltpu.sync_copy(p_hbm.at[0, 0], sa)
        pltpu.sync_copy(p_hbm.at[1, 0], t0)
        pltpu.sync_copy(pa_hbm, t1)

        def ca(i, carry):
            sl = pl.ds(i * L, L)
            sa[sl] = (sa[sl] + t0[sl] + t1[sl]) * dv[sl]
            return carry

        lax.fori_loop(0, npad // L, ca, 0)
        pltpu.sync_copy(p_hbm.at[0, 1], sb)
        pltpu.sync_copy(p_hbm.at[1, 1], t0)
        pltpu.sync_copy(pb_hbm, t1)

        def cb(i, carry):
            sl = pl.ds(i * L, L)
            sb[sl] = (sb[sl] + t0[sl] + t1[sl]) * dv[sl]
            return carry

        lax.fori_loop(0, npad // L, cb, 0)
        pltpu.sync_copy(src_hbm.at[wid], srcv)
        pltpu.sync_copy(dst_hbm.at[wid], dstv)
        pltpu.sync_copy(c_hbm.at[wid], cv)

        def gat(i, carry):
            sl = pl.ds(i * L, L)
            ia = srcv[sl]
            ib = dstv[sl]
            ov[sl] = (plsc.load_gather(sa, [ia]) + plsc.load_gather(sb, [ib])
                      + cv[sl])
            return carry

        lax.fori_loop(0, epw // L, gat, 0)
        pltpu.sync_copy(ov, out_hbm.at[wid])

    return final_kernel(P, pa, pb, dinv, srcf, dstf, cflat)


def _tc_edge(ea, We1p, be1p, wcp, consts, be):
    epad, de = ea.shape
    nblk = epad // be

    def body(ea_ref, w_ref, b_ref, wc_ref, k_ref, out_ref):
        t = jnp.dot(ea_ref[...], w_ref[...], preferred_element_type=f32)
        t = jnp.maximum(t + b_ref[...], 0.0)
        out_ref[0, 0, :] = jnp.sum(t * wc_ref[...], axis=1) + k_ref[0, 0]

    return pl.pallas_call(
        body,
        grid=(nblk,),
        in_specs=[
            pl.BlockSpec((be, de), lambda i: (i, 0)),
            pl.BlockSpec((de, HP), lambda i: (0, 0)),
            pl.BlockSpec((1, HP), lambda i: (0, 0)),
            pl.BlockSpec((1, HP), lambda i: (0, 0)),
            pl.BlockSpec(memory_space=pltpu.SMEM),
        ],
        out_specs=pl.BlockSpec((1, 1, be), lambda i: (i, 0, 0)),
        out_shape=jax.ShapeDtypeStruct((nblk, 1, be), f32),
    )(ea, We1p, be1p, wcp, consts)


def _tc_node1(xp, W1p, deg2, npad, r):
    nblk = npad // r
    d = xp.shape[1]

    def body(x_ref, w_ref, deg_ref, xs_ref, dinv_ref):
        deg = deg_ref[0, :] + deg_ref[1, :] + 1.0
        dinv = lax.rsqrt(deg)
        xw = jnp.dot(x_ref[...], w_ref[...], preferred_element_type=f32)
        xs_ref[...] = xw * dinv[:, None]
        dinv_ref[0, 0, :] = dinv

    return pl.pallas_call(
        body,
        grid=(nblk,),
        in_specs=[
            pl.BlockSpec((r, d), lambda i: (i, 0)),
            pl.BlockSpec((d, HP), lambda i: (0, 0)),
            pl.BlockSpec((2, r), lambda i: (0, i)),
        ],
        out_specs=[
            pl.BlockSpec((r, HP), lambda i: (i, 0)),
            pl.BlockSpec((1, 1, r), lambda i: (i, 0, 0)),
        ],
        out_shape=[
            jax.ShapeDtypeStruct((npad, HP), f32),
            jax.ShapeDtypeStruct((nblk, 1, r), f32),
        ],
    )(xp, W1p, deg2)


def _tc_node2(agg, xs1, dinv2, b1p, w2ab, npad, r):
    nblk = npad // r

    def body(agg_ref, xs_ref, dinv_ref, b_ref, w_ref, pa_ref, pb_ref):
        dinv = dinv_ref[0, 0, :]
        pre = (agg_ref[0] + agg_ref[1] + xs_ref[...]) * dinv[:, None]
        h1 = jnp.maximum(pre + b_ref[...], 0.0)
        pa_ref[0, 0, :] = dinv * jnp.sum(h1 * w_ref[0:1, :], axis=1)
        pb_ref[0, 0, :] = dinv * jnp.sum(h1 * w_ref[1:2, :], axis=1)

    return pl.pallas_call(
        body,
        grid=(nblk,),
        in_specs=[
            pl.BlockSpec((2, r, HP), lambda i: (0, i, 0)),
            pl.BlockSpec((r, HP), lambda i: (i, 0)),
            pl.BlockSpec((1, 1, r), lambda i: (i, 0, 0)),
            pl.BlockSpec((1, HP), lambda i: (0, 0)),
            pl.BlockSpec((2, HP), lambda i: (0, 0)),
        ],
        out_specs=[
            pl.BlockSpec((1, 1, r), lambda i: (i, 0, 0)),
            pl.BlockSpec((1, 1, r), lambda i: (i, 0, 0)),
        ],
        out_shape=[
            jax.ShapeDtypeStruct((nblk, 1, r), f32),
            jax.ShapeDtypeStruct((nblk, 1, r), f32),
        ],
    )(agg, xs1, dinv2, b1p, w2ab)


def kernel(x, edge_index, edge_attr, W1, b1, W2, b2, We1, be1, We2, be2, Wf, bf):
    n, d = x.shape
    e = edge_index.shape[1]
    h = W1.shape[1]

    r = 1024
    npad = _round_up(n + 1, r)
    nb = _round_up(e, NW * B) // (NW * B)
    epw = nb * B
    epad = NW * epw
    be = max(NW * B, 2560)
    while epad % be:
        be //= 2

    pad_h = HP - h
    W1p = jnp.pad(W1, ((0, 0), (0, pad_h)))
    We1p = jnp.pad(We1, ((0, 0), (0, pad_h)))
    be1p = jnp.pad(be1, (0, pad_h)).reshape(1, HP)
    b1p = jnp.pad(b1, (0, pad_h)).reshape(1, HP)
    Wfa, Wfb, Wfc = Wf[:h, 0], Wf[h:2 * h, 0], Wf[2 * h:, 0]
    w2ab = jnp.pad(jnp.stack([W2 @ Wfa, W2 @ Wfb]), ((0, 0), (0, pad_h)))
    wcp = jnp.pad(We2 @ Wfc, (0, pad_h)).reshape(1, HP)
    consts = (be2 @ Wfc + bf[0] + b2 @ Wfa + b2 @ Wfb).reshape(1, 1)

    xp = jnp.pad(x, ((0, npad - n), (0, 0)))
    eap = jnp.pad(edge_attr, ((0, epad - e), (0, 0)))
    src = jnp.pad(edge_index[0], (0, epad - e), constant_values=n)
    dst = jnp.pad(edge_index[1], (0, epad - e), constant_values=n)
    src3 = src.reshape(NW, nb, B)
    dst3 = dst.reshape(NW, nb, B)
    srcf = src.reshape(NW, epw)
    dstf = dst.reshape(NW, epw)

    deg2 = _sc_deg(dst3, npad, nb)
    cflat = _tc_edge(eap, We1p, be1p, wcp, consts, be).reshape(NW, epw)
    xs1, dinv2 = _tc_node1(xp, W1p, deg2, npad, r)
    agg = _sc_conv(xs1, src3, dst3, npad, nb)
    pa2, pb2 = _tc_node2(agg, xs1, dinv2, b1p, w2ab, npad, r)
    P = _sc_hist2(pa2.reshape(npad), pb2.reshape(npad), srcf, dst3, npad, nb,
                  epw)
    pred = _sc_final(P, pa2.reshape(npad), pb2.reshape(npad),
                     dinv2.reshape(npad), srcf, dstf, cflat, npad, epw)
    return pred.reshape(epad)[:e]


# speedup vs baseline: 22.7636x; 1.0390x over previous
import functools

import jax
import jax.numpy as jnp
from jax import lax
from jax.experimental import pallas as pl
from jax.experimental.pallas import tpu as pltpu
from jax.experimental.pallas import tpu_sc as plsc

NC = 2
NS = 16
NW = NC * NS
L = 16
HP = 32
B = 80

f32 = jnp.float32
i32 = jnp.int32


def _mesh():
    return plsc.VectorSubcoreMesh(
        core_axis_name="c", subcore_axis_name="s", num_cores=NC, num_subcores=NS
    )


def _round_up(a, m):
    return (a + m - 1) // m * m


def _sc_deg(dst3, npad, nb):
    nps = npad // NS

    @functools.partial(
        pl.kernel,
        out_type=jax.ShapeDtypeStruct((NC, npad), f32),
        mesh=_mesh(),
        scratch_types=[
            pltpu.VMEM((nb, B), i32),
            pltpu.VMEM((B,), f32),
            pltpu.VMEM((nps,), f32),
            pltpu.VMEM_SHARED((npad,), f32),
        ],
    )
    def deg_kernel(dst_hbm, out_hbm, dstv, ones_v, stage, deg_sh):
        c = lax.axis_index("c")
        s = lax.axis_index("s")
        wid = c * NS + s
        pltpu.sync_copy(dst_hbm.at[wid], dstv)
        one = jnp.ones((L,), f32)
        zero = jnp.zeros((L,), f32)

        def fill_ones(i, carry):
            ones_v[pl.ds(i * L, L)] = one
            return carry

        lax.fori_loop(0, B // L, fill_ones, 0)

        def fill_zero(i, carry):
            stage[pl.ds(i * L, L)] = zero
            return carry

        lax.fori_loop(0, nps // L, fill_zero, 0)
        pltpu.sync_copy(stage, deg_sh.at[pl.ds(s * nps, nps)])
        plsc.subcore_barrier()

        def blk(j, carry):
            pltpu.sync_copy(ones_v, deg_sh.at[dstv.at[j]], add=True)
            return carry

        lax.fori_loop(0, nb, blk, 0)
        plsc.subcore_barrier()
        pltpu.sync_copy(deg_sh.at[pl.ds(s * nps, nps)], stage)
        pltpu.sync_copy(stage, out_hbm.at[c, pl.ds(s * nps, nps)])

    return deg_kernel(dst3)


def _sc_conv(table, src3, dst3, npad, nb):
    nps = npad // NS

    @functools.partial(
        pl.kernel,
        out_type=jax.ShapeDtypeStruct((NC, npad, HP), f32),
        mesh=_mesh(),
        scratch_types=[
            pltpu.VMEM((nb, B), i32),
            pltpu.VMEM((nb, B), i32),
            pltpu.VMEM((B, HP), f32),
            pltpu.VMEM((B, HP), f32),
            pltpu.VMEM((nps, HP), f32),
            pltpu.VMEM_SHARED((npad, HP), f32),
            pltpu.SemaphoreType.DMA,
            pltpu.SemaphoreType.DMA,
        ],
        compiler_params=pltpu.CompilerParams(use_tc_tiling_on_sc=False),
    )
    def conv_kernel(tab_hbm, src_hbm, dst_hbm, out_hbm, srcv, dstv, rows0, rows1,
                    stage, agg_sh, sem0, sem1):
        c = lax.axis_index("c")
        s = lax.axis_index("s")
        wid = c * NS + s
        pltpu.sync_copy(src_hbm.at[wid], srcv)
        pltpu.sync_copy(dst_hbm.at[wid], dstv)
        zero = jnp.zeros((L,), f32)

        def fill_zero(rr, carry):
            stage[rr, pl.ds(0, L)] = zero
            stage[rr, pl.ds(L, L)] = zero
            return carry

        lax.fori_loop(0, nps, fill_zero, 0)
        pltpu.sync_copy(stage, agg_sh.at[pl.ds(s * nps, nps)])
        plsc.subcore_barrier()

        def gat(j, rows, sem):
            return pltpu.make_async_copy(tab_hbm.at[srcv.at[j]], rows, sem)

        gat(0, rows0, sem0).start()

        def blk(jj, carry):
            j = jj * 2
            gat(j, rows0, sem0).wait()
            gat(j + 1, rows1, sem1).start()
            pltpu.sync_copy(rows0, agg_sh.at[dstv.at[j]], add=True)
            gat(j + 1, rows1, sem1).wait()
            gat(j + 2, rows0, sem0).start()
            pltpu.sync_copy(rows1, agg_sh.at[dstv.at[j + 1]], add=True)
            return carry

        npair = (nb - 1) // 2
        lax.fori_loop(0, npair, blk, 0)

        def tail(j, carry):
            gat(j, rows0, sem0).wait()
            pltpu.sync_copy(rows0, agg_sh.at[dstv.at[j]], add=True)
            gat(j + 1, rows0, sem0).start()
            return carry

        lax.fori_loop(2 * npair, nb - 1, tail, 0)
        gat(nb - 1, rows0, sem0).wait()
        pltpu.sync_copy(rows0, agg_sh.at[dstv.at[nb - 1]], add=True)
        plsc.subcore_barrier()
        pltpu.sync_copy(agg_sh.at[pl.ds(s * nps, nps)], stage)
        pltpu.sync_copy(stage, out_hbm.at[c, pl.ds(s * nps, nps)])

    return conv_kernel(table, src3, dst3)


def _sc_hist2(pa, pb, srcf, dst3, npad, nb, epw):
    nps = npad // NS

    @functools.partial(
        pl.kernel,
        out_type=jax.ShapeDtypeStruct((NC, 2, npad), f32),
        mesh=_mesh(),
        scratch_types=[
            pltpu.VMEM((npad,), f32),
            pltpu.VMEM((npad,), f32),
            pltpu.VMEM((epw,), i32),
            pltpu.VMEM((nb, B), i32),
            pltpu.VMEM((epw,), f32),
            pltpu.VMEM((epw,), f32),
            pltpu.VMEM((nps,), f32),
            pltpu.VMEM_SHARED((npad,), f32),
            pltpu.VMEM_SHARED((npad,), f32),
        ],
        compiler_params=pltpu.CompilerParams(needs_layout_passes=False),
    )
    def hist_kernel(pa_hbm, pb_hbm, src_hbm, dst_hbm, out_hbm, pa_t, pb_t,
                    srcv, dstv, va, vb, stage, sh_a, sh_b):
        c = lax.axis_index("c")
        s = lax.axis_index("s")
        wid = c * NS + s
        pltpu.sync_copy(pa_hbm, pa_t)
        pltpu.sync_copy(pb_hbm, pb_t)
        pltpu.sync_copy(src_hbm.at[wid], srcv)
        pltpu.sync_copy(dst_hbm.at[wid], dstv)
        zero = jnp.zeros((L,), f32)

        def fill_zero(i, carry):
            stage[pl.ds(i * L, L)] = zero
            return carry

        lax.fori_loop(0, nps // L, fill_zero, 0)
        pltpu.sync_copy(stage, sh_a.at[pl.ds(s * nps, nps)])
        pltpu.sync_copy(stage, sh_b.at[pl.ds(s * nps, nps)])
        plsc.subcore_barrier()

        def gat(i, carry):
            sl = pl.ds(i * L, L)
            idx = srcv[sl]
            va[sl] = plsc.load_gather(pa_t, [idx])
            vb[sl] = plsc.load_gather(pb_t, [idx])
            return carry

        lax.fori_loop(0, epw // L, gat, 0)

        def blk(j, carry):
            sl = pl.ds(j * B, B)
            pltpu.sync_copy(va.at[sl], sh_a.at[dstv.at[j]], add=True)
            pltpu.sync_copy(vb.at[sl], sh_b.at[dstv.at[j]], add=True)
            return carry

        lax.fori_loop(0, nb, blk, 0)
        plsc.subcore_barrier()
        sl = pl.ds(s * nps, nps)
        pltpu.sync_copy(sh_a.at[sl], stage)
        pltpu.sync_copy(stage, out_hbm.at[c, 0, sl])
        pltpu.sync_copy(sh_b.at[sl], stage)
        pltpu.sync_copy(stage, out_hbm.at[c, 1, sl])

    return hist_kernel(pa, pb, srcf, dst3)


def _sc_final(P, pa, pb, dinv, srcf, dstf, cflat, npad, epw):
    @functools.partial(
        pl.kernel,
        out_type=jax.ShapeDtypeStruct((NW, epw), f32),
        mesh=_mesh(),
        scratch_types=[
            pltpu.VMEM((npad,), f32),
            pltpu.VMEM((npad,), f32),
            pltpu.VMEM((npad,), f32),
            pltpu.VMEM((npad,), f32),
            pltpu.VMEM((npad,), f32),
            pltpu.VMEM((epw,), i32),
            pltpu.VMEM((epw,), i32),
            pltpu.VMEM((epw,), f32),
            pltpu.VMEM((epw,), f32),
        ],
        compiler_params=pltpu.CompilerParams(needs_layout_passes=False),
    )
    def final_kernel(p_hbm, pa_hbm, pb_hbm, dinv_hbm, src_hbm, dst_hbm, c_hbm,
                     out_hbm, sa, sb, t0, t1, dv, srcv, dstv, cv, ov):
        c = lax.axis_index("c")
        s = lax.axis_index("s")
        wid = c * NS + s
        pltpu.sync_copy(dinv_hbm, dv)
        pltpu.sync_copy(p_hbm.at[0, 0], sa)
        pltpu.sync_copy(p_hbm.at[1, 0], t0)
        pltpu.sync_copy(pa_hbm, t1)

        def ca(i, carry):
            sl = pl.ds(i * L, L)
            sa[sl] = (sa[sl] + t0[sl] + t1[sl]) * dv[sl]
            return carry

        lax.fori_loop(0, npad // L, ca, 0)
        pltpu.sync_copy(p_hbm.at[0, 1], sb)
        pltpu.sync_copy(p_hbm.at[1, 1], t0)
        pltpu.sync_copy(pb_hbm, t1)

        def cb(i, carry):
            sl = pl.ds(i * L, L)
            sb[sl] = (sb[sl] + t0[sl] + t1[sl]) * dv[sl]
            return carry

        lax.fori_loop(0, npad // L, cb, 0)
        pltpu.sync_copy(src_hbm.at[wid], srcv)
        pltpu.sync_copy(dst_hbm.at[wid], dstv)
        pltpu.sync_copy(c_hbm.at[wid], cv)

        def gat(i, carry):
            sl = pl.ds(i * L, L)
            ia = srcv[sl]
            ib = dstv[sl]
            ov[sl] = (plsc.load_gather(sa, [ia]) + plsc.load_gather(sb, [ib])
                      + cv[sl])
            return carry

        lax.fori_loop(0, epw // L, gat, 0)
        pltpu.sync_copy(ov, out_hbm.at[wid])

    return final_kernel(P, pa, pb, dinv, srcf, dstf, cflat)


def _tc_edge(ea, We1p, be1p, wcp, consts, be):
    epad, de = ea.shape
    nblk = epad // be

    def body(ea_ref, w_ref, b_ref, wc_ref, k_ref, out_ref):
        t = jnp.dot(ea_ref[...], w_ref[...], preferred_element_type=f32)
        t = jnp.maximum(t + b_ref[...], 0.0)
        out_ref[0, 0, :] = jnp.sum(t * wc_ref[...], axis=1) + k_ref[0, 0]

    return pl.pallas_call(
        body,
        grid=(nblk,),
        in_specs=[
            pl.BlockSpec((be, de), lambda i: (i, 0)),
            pl.BlockSpec((de, HP), lambda i: (0, 0)),
            pl.BlockSpec((1, HP), lambda i: (0, 0)),
            pl.BlockSpec((1, HP), lambda i: (0, 0)),
            pl.BlockSpec(memory_space=pltpu.SMEM),
        ],
        out_specs=pl.BlockSpec((1, 1, be), lambda i: (i, 0, 0)),
        out_shape=jax.ShapeDtypeStruct((nblk, 1, be), f32),
    )(ea, We1p, be1p, wcp, consts)


def _tc_node1(xp, W1p, deg2, npad, r):
    nblk = npad // r
    d = xp.shape[1]

    def body(x_ref, w_ref, deg_ref, xs_ref, dinv_ref):
        deg = deg_ref[0, :] + deg_ref[1, :] + 1.0
        dinv = lax.rsqrt(deg)
        xw = jnp.dot(x_ref[...], w_ref[...], preferred_element_type=f32)
        xs_ref[...] = xw * dinv[:, None]
        dinv_ref[0, 0, :] = dinv

    return pl.pallas_call(
        body,
        grid=(nblk,),
        in_specs=[
            pl.BlockSpec((r, d), lambda i: (i, 0)),
            pl.BlockSpec((d, HP), lambda i: (0, 0)),
            pl.BlockSpec((2, r), lambda i: (0, i)),
        ],
        out_specs=[
            pl.BlockSpec((r, HP), lambda i: (i, 0)),
            pl.BlockSpec((1, 1, r), lambda i: (i, 0, 0)),
        ],
        out_shape=[
            jax.ShapeDtypeStruct((npad, HP), f32),
            jax.ShapeDtypeStruct((nblk, 1, r), f32),
        ],
    )(xp, W1p, deg2)


def _tc_node2(agg, xs1, dinv2, b1p, w2ab, npad, r):
    nblk = npad // r

    def body(agg_ref, xs_ref, dinv_ref, b_ref, w_ref, pa_ref, pb_ref):
        dinv = dinv_ref[0, 0, :]
        pre = (agg_ref[0] + agg_ref[1] + xs_ref[...]) * dinv[:, None]
        h1 = jnp.maximum(pre + b_ref[...], 0.0)
        pa_ref[0, 0, :] = dinv * jnp.sum(h1 * w_ref[0:1, :], axis=1)
        pb_ref[0, 0, :] = dinv * jnp.sum(h1 * w_ref[1:2, :], axis=1)

    return pl.pallas_call(
        body,
        grid=(nblk,),
        in_specs=[
            pl.BlockSpec((2, r, HP), lambda i: (0, i, 0)),
            pl.BlockSpec((r, HP), lambda i: (i, 0)),
            pl.BlockSpec((1, 1, r), lambda i: (i, 0, 0)),
            pl.BlockSpec((1, HP), lambda i: (0, 0)),
            pl.BlockSpec((2, HP), lambda i: (0, 0)),
        ],
        out_specs=[
            pl.BlockSpec((1, 1, r), lambda i: (i, 0, 0)),
            pl.BlockSpec((1, 1, r), lambda i: (i, 0, 0)),
        ],
        out_shape=[
            jax.ShapeDtypeStruct((nblk, 1, r), f32),
            jax.ShapeDtypeStruct((nblk, 1, r), f32),
        ],
    )(agg, xs1, dinv2, b1p, w2ab)


def kernel(x, edge_index, edge_attr, W1, b1, W2, b2, We1, be1, We2, be2, Wf, bf):
    n, d = x.shape
    e = edge_index.shape[1]
    h = W1.shape[1]

    r = 1024
    npad = _round_up(n + 1, r)
    nb = _round_up(e, NW * B) // (NW * B)
    epw = nb * B
    epad = NW * epw
    be = max(NW * B, 2560)
    while epad % be:
        be //= 2

    pad_h = HP - h
    W1p = jnp.pad(W1, ((0, 0), (0, pad_h)))
    We1p = jnp.pad(We1, ((0, 0), (0, pad_h)))
    be1p = jnp.pad(be1, (0, pad_h)).reshape(1, HP)
    b1p = jnp.pad(b1, (0, pad_h)).reshape(1, HP)
    Wfa, Wfb, Wfc = Wf[:h, 0], Wf[h:2 * h, 0], Wf[2 * h:, 0]
    w2ab = jnp.pad(jnp.stack([W2 @ Wfa, W2 @ Wfb]), ((0, 0), (0, pad_h)))
    wcp = jnp.pad(We2 @ Wfc, (0, pad_h)).reshape(1, HP)
    consts = (be2 @ Wfc + bf[0] + b2 @ Wfa + b2 @ Wfb).reshape(1, 1)

    xp = jnp.pad(x, ((0, npad - n), (0, 0)))
    eap = jnp.pad(edge_attr, ((0, epad - e), (0, 0)))
    src = jnp.pad(edge_index[0], (0, epad - e), constant_values=n)
    dst = jnp.pad(edge_index[1], (0, epad - e), constant_values=n)
    src3 = src.reshape(NW, nb, B)
    dst3 = dst.reshape(NW, nb, B)
    srcf = src.reshape(NW, epw)
    dstf = dst.reshape(NW, epw)

    deg2 = _sc_deg(dst3, npad, nb)
    cflat = _tc_edge(eap, We1p, be1p, wcp, consts, be).reshape(NW, epw)
    xs1, dinv2 = _tc_node1(xp, W1p, deg2, npad, r)
    agg = _sc_conv(xs1, src3, dst3, npad, nb)
    pa2, pb2 = _tc_node2(agg, xs1, dinv2, b1p, w2ab, npad, r)
    P = _sc_hist2(pa2.reshape(npad), pb2.reshape(npad), srcf, dst3, npad, nb,
                  epw)
    pred = _sc_final(P, pa2.reshape(npad), pb2.reshape(npad),
                     dinv2.reshape(npad), srcf, dstf, cflat, npad, epw)
    return pred.reshape(epad)[:e]


# speedup vs baseline: 25.2747x; 1.1103x over previous
import functools

import jax
import jax.numpy as jnp
from jax import lax
from jax.experimental import pallas as pl
from jax.experimental.pallas import tpu as pltpu
from jax.experimental.pallas import tpu_sc as plsc

NC = 2
NS = 16
NW = NC * NS
L = 16
HP = 32
B = 80

f32 = jnp.float32
i32 = jnp.int32


def _mesh():
    return plsc.VectorSubcoreMesh(
        core_axis_name="c", subcore_axis_name="s", num_cores=NC, num_subcores=NS
    )


def _round_up(a, m):
    return (a + m - 1) // m * m


def _rsqrt_newton(d):
    y = plsc.bitcast(jnp.int32(0x5F3759DF) - (plsc.bitcast(d, i32) >> 1), f32)
    for _ in range(3):
        y = y * (1.5 - 0.5 * d * y * y)
    return y


def _sc_conv(xw, src3, dst3, npad, nb):
    nps = npad // NS

    @functools.partial(
        pl.kernel,
        out_type=[
            jax.ShapeDtypeStruct((NC, npad, HP), f32),
            jax.ShapeDtypeStruct((NC, npad), f32),
        ],
        mesh=_mesh(),
        scratch_types=[
            pltpu.VMEM((nb, B), i32),
            pltpu.VMEM((nb, B), i32),
            pltpu.VMEM((NC, nb, B), i32),
            pltpu.VMEM((B,), f32),
            pltpu.VMEM((nps,), f32),
            pltpu.VMEM((nps, HP), f32),
            pltpu.VMEM((B, HP), f32),
            pltpu.VMEM((B, HP), f32),
            pltpu.VMEM((nps, HP), f32),
            pltpu.VMEM_SHARED((npad,), f32),
            pltpu.VMEM_SHARED((npad, HP), f32),
            pltpu.VMEM_SHARED((npad, HP), f32),
            pltpu.SemaphoreType.DMA,
            pltpu.SemaphoreType.DMA,
        ],
        compiler_params=pltpu.CompilerParams(
            use_tc_tiling_on_sc=False, needs_layout_passes=False),
    )
    def conv_kernel(xw_hbm, src_hbm, dst_hbm, agg_out, dinv_out,
                    srcv, dstv, dsth, ones_v, dvs, xwb, rows0, rows1, stage,
                    deg_sh, tab_sh, agg_sh, sem0, sem1):
        c = lax.axis_index("c")
        s = lax.axis_index("s")
        wid = c * NS + s
        nsl = pl.ds(s * nps, nps)
        pltpu.sync_copy(src_hbm.at[wid], srcv)
        pltpu.sync_copy(dst_hbm.at[wid], dstv)
        pltpu.sync_copy(dst_hbm.at[pl.ds(NC * s, NC)], dsth)
        one = jnp.ones((L,), f32)
        zero = jnp.zeros((L,), f32)

        def fill_ones(i, carry):
            ones_v[pl.ds(i * L, L)] = one
            return carry

        lax.fori_loop(0, B // L, fill_ones, 0)

        def zero_dvs(i, carry):
            dvs[pl.ds(i * L, L)] = zero
            return carry

        lax.fori_loop(0, nps // L, zero_dvs, 0)

        def zero_stage(rr, carry):
            stage[rr, pl.ds(0, L)] = zero
            stage[rr, pl.ds(L, L)] = zero
            return carry

        lax.fori_loop(0, nps, zero_stage, 0)
        pltpu.sync_copy(dvs, deg_sh.at[nsl])
        pltpu.sync_copy(stage, agg_sh.at[nsl])
        plsc.subcore_barrier()

        for q in range(NC):
            def deg_blk(j, carry, q=q):
                pltpu.sync_copy(ones_v, deg_sh.at[dsth.at[q, j]], add=True)
                return carry

            lax.fori_loop(0, nb, deg_blk, 0)
        plsc.subcore_barrier()

        pltpu.sync_copy(deg_sh.at[nsl], dvs)
        pltpu.sync_copy(xw_hbm.at[pl.ds(s * nps, nps)], xwb)

        def mk_dinv(i, carry):
            sl = pl.ds(i * L, L)
            dvs[sl] = _rsqrt_newton(dvs[sl] + 1.0)
            return carry

        lax.fori_loop(0, nps // L, mk_dinv, 0)

        def scale_row(rr, carry):
            dvec = plsc.load_gather(dvs, [jnp.full((L,), rr, i32)])
            xwb[rr, pl.ds(0, L)] = xwb[rr, pl.ds(0, L)] * dvec
            xwb[rr, pl.ds(L, L)] = xwb[rr, pl.ds(L, L)] * dvec
            return carry

        lax.fori_loop(0, nps, scale_row, 0)
        pltpu.sync_copy(xwb, tab_sh.at[nsl])
        pltpu.sync_copy(dvs, dinv_out.at[c, nsl])
        plsc.subcore_barrier()

        def gat(j, rows, sem):
            return pltpu.make_async_copy(tab_sh.at[srcv.at[j]], rows, sem)

        gat(0, rows0, sem0).start()

        def blk(jj, carry):
            j = jj * 2
            gat(j, rows0, sem0).wait()
            gat(j + 1, rows1, sem1).start()
            pltpu.sync_copy(rows0, agg_sh.at[dstv.at[j]], add=True)
            gat(j + 1, rows1, sem1).wait()
            gat(j + 2, rows0, sem0).start()
            pltpu.sync_copy(rows1, agg_sh.at[dstv.at[j + 1]], add=True)
            return carry

        npair = (nb - 1) // 2
        lax.fori_loop(0, npair, blk, 0)

        def tail(j, carry):
            gat(j, rows0, sem0).wait()
            pltpu.sync_copy(rows0, agg_sh.at[dstv.at[j]], add=True)
            gat(j + 1, rows0, sem0).start()
            return carry

        lax.fori_loop(2 * npair, nb - 1, tail, 0)
        gat(nb - 1, rows0, sem0).wait()
        pltpu.sync_copy(rows0, agg_sh.at[dstv.at[nb - 1]], add=True)
        plsc.subcore_barrier()
        pltpu.sync_copy(agg_sh.at[nsl], stage)
        pltpu.sync_copy(stage, agg_out.at[c, nsl])

    return conv_kernel(xw, src3, dst3)


def _sc_tail(pa, pb, dinv, src3, dst3, cflat, npad, nb, epw):
    nps = npad // NS

    @functools.partial(
        pl.kernel,
        out_type=jax.ShapeDtypeStruct((NW, epw), f32),
        mesh=_mesh(),
        scratch_types=[
            pltpu.VMEM((npad,), f32),
            pltpu.VMEM((npad,), f32),
            pltpu.VMEM((nb, B), i32),
            pltpu.VMEM((nb, B), i32),
            pltpu.VMEM((B,), f32),
            pltpu.VMEM((B,), f32),
            pltpu.VMEM((nps,), f32),
            pltpu.VMEM((nps,), f32),
            pltpu.VMEM((epw,), f32),
            pltpu.VMEM((epw,), f32),
            pltpu.VMEM_SHARED((npad,), f32),
            pltpu.VMEM_SHARED((npad,), f32),
        ],
        compiler_params=pltpu.CompilerParams(needs_layout_passes=False),
    )
    def tail_kernel(pa_hbm, pb_hbm, dinv_hbm, src_hbm, dst_hbm, c_hbm,
                    out_hbm, sa_t, sb_t, srcq, dstq, g80a, g80b, dv_s, ha_s,
                    cg, ov, sh_a, sh_b):
        c = lax.axis_index("c")
        s = lax.axis_index("s")
        wid = c * NS + s
        nsl = pl.ds(s * nps, nps)
        pltpu.sync_copy(pa_hbm, sa_t)
        pltpu.sync_copy(pb_hbm, sb_t)
        pltpu.sync_copy(dinv_hbm.at[nsl], dv_s)
        zero = jnp.zeros((L,), f32)

        def zero_ha(i, carry):
            ha_s[pl.ds(i * L, L)] = zero
            return carry

        lax.fori_loop(0, nps // L, zero_ha, 0)
        pltpu.sync_copy(ha_s, sh_a.at[nsl])
        pltpu.sync_copy(ha_s, sh_b.at[nsl])
        plsc.subcore_barrier()

        for q in range(NC):
            pltpu.sync_copy(src_hbm.at[NC * s + q], srcq)
            pltpu.sync_copy(dst_hbm.at[NC * s + q], dstq)

            def hist_blk(j, carry):
                for k in range(B // L):
                    ksl = pl.ds(k * L, L)
                    idx = srcq[j, ksl]
                    g80a[ksl] = plsc.load_gather(sa_t, [idx])
                    g80b[ksl] = plsc.load_gather(sb_t, [idx])
                pltpu.sync_copy(g80a, sh_a.at[dstq.at[j]], add=True)
                pltpu.sync_copy(g80b, sh_b.at[dstq.at[j]], add=True)
                return carry

            lax.fori_loop(0, nb, hist_blk, 0)
        plsc.subcore_barrier()

        pltpu.sync_copy(sh_a.at[nsl], ha_s)

        def mk_sa(i, carry):
            sl = pl.ds(i * L, L)
            gsl = pl.ds(s * nps + i * L, L)
            ha_s[sl] = dv_s[sl] * (ha_s[sl] + sa_t[gsl])
            return carry

        lax.fori_loop(0, nps // L, mk_sa, 0)
        pltpu.sync_copy(ha_s, sh_a.at[nsl])
        pltpu.sync_copy(sh_b.at[nsl], ha_s)

        def mk_sb(i, carry):
            sl = pl.ds(i * L, L)
            gsl = pl.ds(s * nps + i * L, L)
            ha_s[sl] = dv_s[sl] * (ha_s[sl] + sb_t[gsl])
            return carry

        lax.fori_loop(0, nps // L, mk_sb, 0)
        pltpu.sync_copy(ha_s, sh_b.at[nsl])
        plsc.subcore_barrier()
        pltpu.sync_copy(sh_a, sa_t)
        pltpu.sync_copy(sh_b, sb_t)

        pltpu.sync_copy(src_hbm.at[wid], srcq)
        pltpu.sync_copy(dst_hbm.at[wid], dstq)
        pltpu.sync_copy(c_hbm.at[wid], cg)

        def gatf(j, carry):
            for k in range(B // L):
                ksl = pl.ds(k * L, L)
                sl = pl.ds(j * B + k * L, L)
                ov[sl] = (plsc.load_gather(sa_t, [srcq[j, ksl]])
                          + plsc.load_gather(sb_t, [dstq[j, ksl]]) + cg[sl])
            return carry

        lax.fori_loop(0, nb, gatf, 0)
        pltpu.sync_copy(ov, out_hbm.at[wid])

    return tail_kernel(pa, pb, dinv, src3, dst3, cflat)


def _tc_edge(ea, We1p, be1p, wcp, consts, be):
    epad, de = ea.shape
    nblk = epad // be

    def body(ea_ref, w_ref, b_ref, wc_ref, k_ref, out_ref):
        t = jnp.dot(ea_ref[...], w_ref[...], preferred_element_type=f32)
        t = jnp.maximum(t + b_ref[...], 0.0)
        out_ref[0, 0, :] = jnp.sum(t * wc_ref[...], axis=1) + k_ref[0, 0]

    return pl.pallas_call(
        body,
        grid=(nblk,),
        in_specs=[
            pl.BlockSpec((be, de), lambda i: (i, 0)),
            pl.BlockSpec((de, HP), lambda i: (0, 0)),
            pl.BlockSpec((1, HP), lambda i: (0, 0)),
            pl.BlockSpec((1, HP), lambda i: (0, 0)),
            pl.BlockSpec(memory_space=pltpu.SMEM),
        ],
        out_specs=pl.BlockSpec((1, 1, be), lambda i: (i, 0, 0)),
        out_shape=jax.ShapeDtypeStruct((nblk, 1, be), f32),
    )(ea, We1p, be1p, wcp, consts)


def _tc_xw(xp, W1p, npad, r):
    nblk = npad // r
    d = xp.shape[1]

    def body(x_ref, w_ref, xw_ref):
        xw_ref[...] = jnp.dot(x_ref[...], w_ref[...],
                              preferred_element_type=f32)

    return pl.pallas_call(
        body,
        grid=(nblk,),
        in_specs=[
            pl.BlockSpec((r, d), lambda i: (i, 0)),
            pl.BlockSpec((d, HP), lambda i: (0, 0)),
        ],
        out_specs=pl.BlockSpec((r, HP), lambda i: (i, 0)),
        out_shape=jax.ShapeDtypeStruct((npad, HP), f32),
    )(xp, W1p)


def _tc_node2(agg, xw, dinv3, b1p, w2ab, npad, r):
    nblk = npad // r

    def body(agg_ref, xw_ref, dinv_ref, b_ref, w_ref, pa_ref, pb_ref):
        dinv = dinv_ref[0, 0, :]
        xs = xw_ref[...] * dinv[:, None]
        pre = (agg_ref[0] + agg_ref[1] + xs) * dinv[:, None]
        h1 = jnp.maximum(pre + b_ref[...], 0.0)
        pa_ref[0, 0, :] = dinv * jnp.sum(h1 * w_ref[0:1, :], axis=1)
        pb_ref[0, 0, :] = dinv * jnp.sum(h1 * w_ref[1:2, :], axis=1)

    return pl.pallas_call(
        body,
        grid=(nblk,),
        in_specs=[
            pl.BlockSpec((2, r, HP), lambda i: (0, i, 0)),
            pl.BlockSpec((r, HP), lambda i: (i, 0)),
            pl.BlockSpec((1, 1, r), lambda i: (i, 0, 0)),
            pl.BlockSpec((1, HP), lambda i: (0, 0)),
            pl.BlockSpec((2, HP), lambda i: (0, 0)),
        ],
        out_specs=[
            pl.BlockSpec((1, 1, r), lambda i: (i, 0, 0)),
            pl.BlockSpec((1, 1, r), lambda i: (i, 0, 0)),
        ],
        out_shape=[
            jax.ShapeDtypeStruct((nblk, 1, r), f32),
            jax.ShapeDtypeStruct((nblk, 1, r), f32),
        ],
    )(agg, xw, dinv3, b1p, w2ab)


def kernel(x, edge_index, edge_attr, W1, b1, W2, b2, We1, be1, We2, be2, Wf, bf):
    n, d = x.shape
    e = edge_index.shape[1]
    h = W1.shape[1]

    r = 1024
    npad = _round_up(n + 1, r)
    nb = _round_up(e, NW * B) // (NW * B)
    epw = nb * B
    epad = NW * epw
    be = max(NW * B, 2560)
    while epad % be:
        be //= 2

    pad_h = HP - h
    W1p = jnp.pad(W1, ((0, 0), (0, pad_h)))
    We1p = jnp.pad(We1, ((0, 0), (0, pad_h)))
    be1p = jnp.pad(be1, (0, pad_h)).reshape(1, HP)
    b1p = jnp.pad(b1, (0, pad_h)).reshape(1, HP)
    Wfa, Wfb, Wfc = Wf[:h, 0], Wf[h:2 * h, 0], Wf[2 * h:, 0]
    w2ab = jnp.pad(jnp.stack([W2 @ Wfa, W2 @ Wfb]), ((0, 0), (0, pad_h)))
    wcp = jnp.pad(We2 @ Wfc, (0, pad_h)).reshape(1, HP)
    consts = (be2 @ Wfc + bf[0] + b2 @ Wfa + b2 @ Wfb).reshape(1, 1)

    xp = jnp.pad(x, ((0, npad - n), (0, 0)))
    eap = jnp.pad(edge_attr, ((0, epad - e), (0, 0)))
    src = jnp.pad(edge_index[0], (0, epad - e), constant_values=n)
    dst = jnp.pad(edge_index[1], (0, epad - e), constant_values=n)
    src3 = src.reshape(NW, nb, B)
    dst3 = dst.reshape(NW, nb, B)

    xw = _tc_xw(xp, W1p, npad, r)
    cflat = _tc_edge(eap, We1p, be1p, wcp, consts, be).reshape(NW, epw)
    agg, dinv2 = _sc_conv(xw, src3, dst3, npad, nb)
    dinv = dinv2[0]
    pa2, pb2 = _tc_node2(agg, xw, dinv.reshape(npad // r, 1, r), b1p, w2ab,
                         npad, r)
    pred = _sc_tail(pa2.reshape(npad), pb2.reshape(npad), dinv, src3, dst3,
                    cflat, npad, nb, epw)
    return pred.reshape(epad)[:e]


# speedup vs baseline: 44.0860x; 1.7443x over previous
import functools

import jax
import jax.numpy as jnp
from jax import lax
from jax.experimental import pallas as pl
from jax.experimental.pallas import tpu as pltpu
from jax.experimental.pallas import tpu_sc as plsc

NC = 2
NS = 16
NW = NC * NS
L = 16
HP = 32
B = 80

f32 = jnp.float32
i32 = jnp.int32


def _mesh():
    return plsc.VectorSubcoreMesh(
        core_axis_name="c", subcore_axis_name="s", num_cores=NC, num_subcores=NS
    )


def _round_up(a, m):
    return (a + m - 1) // m * m


def _rsqrt_newton(d):
    y = plsc.bitcast(jnp.int32(0x5F3759DF) - (plsc.bitcast(d, i32) >> 1), f32)
    for _ in range(3):
        y = y * (1.5 - 0.5 * d * y * y)
    return y


def _sc_conv(xw, src3, dst3, npad, nb):
    nps = npad // NS

    @functools.partial(
        pl.kernel,
        out_type=[
            jax.ShapeDtypeStruct((NC, npad, HP), f32),
            jax.ShapeDtypeStruct((NC, npad), f32),
        ],
        mesh=_mesh(),
        scratch_types=[
            pltpu.VMEM((nb, B), i32),
            pltpu.VMEM((nb, B), i32),
            pltpu.VMEM((NC, nb, B), i32),
            pltpu.VMEM((B,), f32),
            pltpu.VMEM((nps,), f32),
            pltpu.VMEM((nps, HP), f32),
            pltpu.VMEM((B, HP), f32),
            pltpu.VMEM((B, HP), f32),
            pltpu.VMEM((nps, HP), f32),
            pltpu.VMEM_SHARED((npad,), f32),
            pltpu.VMEM_SHARED((npad, HP), f32),
            pltpu.VMEM_SHARED((npad, HP), f32),
            pltpu.SemaphoreType.DMA,
            pltpu.SemaphoreType.DMA,
        ],
        compiler_params=pltpu.CompilerParams(
            use_tc_tiling_on_sc=False, needs_layout_passes=False),
    )
    def conv_kernel(xw_hbm, src_hbm, dst_hbm, agg_out, dinv_out,
                    srcv, dstv, dsth, ones_v, dvs, xwb, rows0, rows1, stage,
                    deg_sh, tab_sh, agg_sh, sem0, sem1):
        c = lax.axis_index("c")
        s = lax.axis_index("s")
        wid = c * NS + s
        nsl = pl.ds(s * nps, nps)
        pltpu.sync_copy(src_hbm.at[wid], srcv)
        pltpu.sync_copy(dst_hbm.at[wid], dstv)
        pltpu.sync_copy(dst_hbm.at[pl.ds(NC * s, NC)], dsth)
        one = jnp.ones((L,), f32)
        zero = jnp.zeros((L,), f32)

        def fill_ones(i, carry):
            ones_v[pl.ds(i * L, L)] = one
            return carry

        lax.fori_loop(0, B // L, fill_ones, 0)

        def zero_dvs(i, carry):
            dvs[pl.ds(i * L, L)] = zero
            return carry

        lax.fori_loop(0, nps // L, zero_dvs, 0)

        def zero_stage(rr, carry):
            stage[rr, pl.ds(0, L)] = zero
            stage[rr, pl.ds(L, L)] = zero
            return carry

        lax.fori_loop(0, nps, zero_stage, 0)
        pltpu.sync_copy(dvs, deg_sh.at[nsl])
        pltpu.sync_copy(stage, agg_sh.at[nsl])
        plsc.subcore_barrier()

        for q in range(NC):
            def deg_blk(j, carry, q=q):
                pltpu.sync_copy(ones_v, deg_sh.at[dsth.at[q, j]], add=True)
                return carry

            lax.fori_loop(0, nb, deg_blk, 0)
        plsc.subcore_barrier()

        pltpu.sync_copy(deg_sh.at[nsl], dvs)
        pltpu.sync_copy(xw_hbm.at[pl.ds(s * nps, nps)], xwb)

        def mk_dinv(i, carry):
            sl = pl.ds(i * L, L)
            dvs[sl] = _rsqrt_newton(dvs[sl] + 1.0)
            return carry

        lax.fori_loop(0, nps // L, mk_dinv, 0)

        def scale_row(rr, carry):
            dvec = plsc.load_gather(dvs, [jnp.full((L,), rr, i32)])
            xwb[rr, pl.ds(0, L)] = xwb[rr, pl.ds(0, L)] * dvec
            xwb[rr, pl.ds(L, L)] = xwb[rr, pl.ds(L, L)] * dvec
            return carry

        lax.fori_loop(0, nps, scale_row, 0)
        pltpu.sync_copy(xwb, tab_sh.at[nsl])
        pltpu.sync_copy(dvs, dinv_out.at[c, nsl])
        plsc.subcore_barrier()

        def gat(j, rows, sem):
            return pltpu.make_async_copy(tab_sh.at[srcv.at[j]], rows, sem)

        gat(0, rows0, sem0).start()

        def blk(jj, carry):
            j = jj * 2
            gat(j, rows0, sem0).wait()
            gat(j + 1, rows1, sem1).start()
            pltpu.sync_copy(rows0, agg_sh.at[dstv.at[j]], add=True)
            gat(j + 1, rows1, sem1).wait()
            gat(j + 2, rows0, sem0).start()
            pltpu.sync_copy(rows1, agg_sh.at[dstv.at[j + 1]], add=True)
            return carry

        npair = (nb - 1) // 2
        lax.fori_loop(0, npair, blk, 0)

        def tail(j, carry):
            gat(j, rows0, sem0).wait()
            pltpu.sync_copy(rows0, agg_sh.at[dstv.at[j]], add=True)
            gat(j + 1, rows0, sem0).start()
            return carry

        lax.fori_loop(2 * npair, nb - 1, tail, 0)
        gat(nb - 1, rows0, sem0).wait()
        pltpu.sync_copy(rows0, agg_sh.at[dstv.at[nb - 1]], add=True)
        plsc.subcore_barrier()
        pltpu.sync_copy(agg_sh.at[nsl], stage)
        pltpu.sync_copy(stage, agg_out.at[c, nsl])

    return conv_kernel(xw, src3, dst3)


def _sc_tail(pa, pb, dinv2, src3, dst3, cflat, npad, nb, epw):
    nps = npad // NS

    @functools.partial(
        pl.kernel,
        out_type=jax.ShapeDtypeStruct((NW * epw,), f32),
        mesh=_mesh(),
        scratch_types=[
            pltpu.VMEM((npad,), f32),
            pltpu.VMEM((npad,), f32),
            pltpu.VMEM((nps,), f32),
            pltpu.VMEM((nps,), f32),
            pltpu.VMEM((nb, B), i32),
            pltpu.VMEM((nb, B), i32),
            pltpu.VMEM((B,), f32),
            pltpu.VMEM((B,), f32),
            pltpu.VMEM((nps,), f32),
            pltpu.VMEM((nps,), f32),
            pltpu.VMEM((epw,), f32),
            pltpu.VMEM((epw,), f32),
            pltpu.VMEM_SHARED((npad,), f32),
            pltpu.VMEM_SHARED((npad,), f32),
        ],
        compiler_params=pltpu.CompilerParams(
            use_tc_tiling_on_sc=False, needs_layout_passes=False),
    )
    def tail_kernel(pa_hbm, pb_hbm, dinv_hbm, src_hbm, dst_hbm, c_hbm,
                    out_hbm, sa_t, sb_t, pav, pbv, srcq, dstq, g80a, g80b,
                    dv_s, ha_s, cg, ov, sh_a, sh_b):
        c = lax.axis_index("c")
        s = lax.axis_index("s")
        wid = c * NS + s
        nsl = pl.ds(s * nps, nps)
        pltpu.sync_copy(pa_hbm, sa_t)
        pltpu.sync_copy(pb_hbm, sb_t)
        pltpu.sync_copy(dinv_hbm.at[0, nsl], dv_s)
        pltpu.sync_copy(pa_hbm.at[nsl], pav)
        pltpu.sync_copy(pb_hbm.at[nsl], pbv)
        zero = jnp.zeros((L,), f32)

        def zero_ha(i, carry):
            ha_s[pl.ds(i * L, L)] = zero
            return carry

        lax.fori_loop(0, nps // L, zero_ha, 0)
        pltpu.sync_copy(ha_s, sh_a.at[nsl])
        pltpu.sync_copy(ha_s, sh_b.at[nsl])
        plsc.subcore_barrier()

        for q in range(NC):
            pltpu.sync_copy(src_hbm.at[NC * s + q], srcq)
            pltpu.sync_copy(dst_hbm.at[NC * s + q], dstq)

            def hist_blk(j, carry):
                for k in range(B // L):
                    ksl = pl.ds(k * L, L)
                    idx = srcq[j, ksl]
                    g80a[ksl] = plsc.load_gather(sa_t, [idx])
                    g80b[ksl] = plsc.load_gather(sb_t, [idx])
                pltpu.sync_copy(g80a, sh_a.at[dstq.at[j]], add=True)
                pltpu.sync_copy(g80b, sh_b.at[dstq.at[j]], add=True)
                return carry

            lax.fori_loop(0, nb, hist_blk, 0)
        plsc.subcore_barrier()

        pltpu.sync_copy(sh_a.at[nsl], ha_s)

        def mk_sa(i, carry):
            sl = pl.ds(i * L, L)
            ha_s[sl] = dv_s[sl] * (ha_s[sl] + pav[sl])
            return carry

        lax.fori_loop(0, nps // L, mk_sa, 0)
        pltpu.sync_copy(ha_s, sh_a.at[nsl])
        pltpu.sync_copy(sh_b.at[nsl], ha_s)

        def mk_sb(i, carry):
            sl = pl.ds(i * L, L)
            ha_s[sl] = dv_s[sl] * (ha_s[sl] + pbv[sl])
            return carry

        lax.fori_loop(0, nps // L, mk_sb, 0)
        pltpu.sync_copy(ha_s, sh_b.at[nsl])
        plsc.subcore_barrier()
        pltpu.sync_copy(sh_a, sa_t)
        pltpu.sync_copy(sh_b, sb_t)

        pltpu.sync_copy(src_hbm.at[wid], srcq)
        pltpu.sync_copy(dst_hbm.at[wid], dstq)
        pltpu.sync_copy(c_hbm.at[pl.ds(wid * epw, epw)], cg)

        def gatf(j, carry):
            for k in range(B // L):
                ksl = pl.ds(k * L, L)
                sl = pl.ds(j * B + k * L, L)
                ov[sl] = (plsc.load_gather(sa_t, [srcq[j, ksl]])
                          + plsc.load_gather(sb_t, [dstq[j, ksl]]) + cg[sl])
            return carry

        lax.fori_loop(0, nb, gatf, 0)
        pltpu.sync_copy(ov, out_hbm.at[pl.ds(wid * epw, epw)])

    return tail_kernel(pa, pb, dinv2, src3, dst3, cflat)


def _tc_edge(eaT, We1pT, be1c, wcc, consts, be):
    de, epad = eaT.shape
    nblk = epad // be

    def body(ea_ref, w_ref, b_ref, wc_ref, k_ref, out_ref):
        t = jnp.dot(w_ref[...], ea_ref[...], preferred_element_type=f32,
                    precision=jax.lax.Precision.HIGHEST)
        t = jnp.maximum(t + b_ref[...], 0.0)
        out_ref[0, 0, :] = jnp.sum(t * wc_ref[...], axis=0) + k_ref[0, 0]

    return pl.pallas_call(
        body,
        grid=(nblk,),
        in_specs=[
            pl.BlockSpec((de, be), lambda i: (0, i)),
            pl.BlockSpec((HP, de), lambda i: (0, 0)),
            pl.BlockSpec((HP, 1), lambda i: (0, 0)),
            pl.BlockSpec((HP, 1), lambda i: (0, 0)),
            pl.BlockSpec(memory_space=pltpu.SMEM),
        ],
        out_specs=pl.BlockSpec((1, 1, be), lambda i: (i, 0, 0)),
        out_shape=jax.ShapeDtypeStruct((nblk, 1, be), f32),
    )(eaT, We1pT, be1c, wcc, consts)


def _tc_xw(xp, W1p, npad, r):
    nblk = npad // r
    d = xp.shape[1]

    def body(x_ref, w_ref, xw_ref):
        xw_ref[...] = jnp.dot(x_ref[...], w_ref[...],
                              preferred_element_type=f32)

    return pl.pallas_call(
        body,
        grid=(nblk,),
        in_specs=[
            pl.BlockSpec((r, d), lambda i: (i, 0)),
            pl.BlockSpec((d, HP), lambda i: (0, 0)),
        ],
        out_specs=pl.BlockSpec((r, HP), lambda i: (i, 0)),
        out_shape=jax.ShapeDtypeStruct((npad, HP), f32),
    )(xp, W1p)


def _tc_node2(agg, xw, dinv3, b1p, w2ab, npad, r):
    nblk = npad // r

    def body(agg_ref, xw_ref, dinv_ref, b_ref, w_ref, pa_ref, pb_ref):
        dinv = dinv_ref[0, 0, :]
        xs = xw_ref[...] * dinv[:, None]
        pre = (agg_ref[0] + agg_ref[1] + xs) * dinv[:, None]
        h1 = jnp.maximum(pre + b_ref[...], 0.0)
        pa_ref[0, 0, :] = dinv * jnp.sum(h1 * w_ref[0:1, :], axis=1)
        pb_ref[0, 0, :] = dinv * jnp.sum(h1 * w_ref[1:2, :], axis=1)

    return pl.pallas_call(
        body,
        grid=(nblk,),
        in_specs=[
            pl.BlockSpec((2, r, HP), lambda i: (0, i, 0)),
            pl.BlockSpec((r, HP), lambda i: (i, 0)),
            pl.BlockSpec((1, 1, r), lambda i: (i, 0, 0)),
            pl.BlockSpec((1, HP), lambda i: (0, 0)),
            pl.BlockSpec((2, HP), lambda i: (0, 0)),
        ],
        out_specs=[
            pl.BlockSpec((1, 1, r), lambda i: (i, 0, 0)),
            pl.BlockSpec((1, 1, r), lambda i: (i, 0, 0)),
        ],
        out_shape=[
            jax.ShapeDtypeStruct((nblk, 1, r), f32),
            jax.ShapeDtypeStruct((nblk, 1, r), f32),
        ],
    )(agg, xw, dinv3, b1p, w2ab)


def kernel(x, edge_index, edge_attr, W1, b1, W2, b2, We1, be1, We2, be2, Wf, bf):
    n, d = x.shape
    e = edge_index.shape[1]
    h = W1.shape[1]

    r = 1024
    npad = _round_up(n + 1, r)
    nb = _round_up(e, NW * B) // (NW * B)
    epw = nb * B
    epad = NW * epw
    be = max(NW * B, 2560)
    while epad % be:
        be //= 2

    pad_h = HP - h
    W1p = jnp.pad(W1, ((0, 0), (0, pad_h)))
    We1pT = jnp.pad(We1, ((0, 0), (0, pad_h))).T
    be1c = jnp.pad(be1, (0, pad_h)).reshape(HP, 1)
    Wfa, Wfb, Wfc = Wf[:h, 0], Wf[h:2 * h, 0], Wf[2 * h:, 0]
    b1p = jnp.pad(b1, (0, pad_h)).reshape(1, HP)
    w2ab = jnp.pad(jnp.stack([W2 @ Wfa, W2 @ Wfb]), ((0, 0), (0, pad_h)))
    wcc = jnp.pad(We2 @ Wfc, (0, pad_h)).reshape(HP, 1)
    consts = (be2 @ Wfc + bf[0] + b2 @ Wfa + b2 @ Wfb).reshape(1, 1)

    xp = jnp.pad(x, ((0, npad - n), (0, 0)))
    eaT = jnp.pad(edge_attr, ((0, epad - e), (0, 0))).T
    src = jnp.pad(edge_index[0], (0, epad - e), constant_values=n)
    dst = jnp.pad(edge_index[1], (0, epad - e), constant_values=n)
    src3 = src.reshape(NW, nb, B)
    dst3 = dst.reshape(NW, nb, B)

    xw = _tc_xw(xp, W1p, npad, r)
    cflat = _tc_edge(eaT, We1pT, be1c, wcc, consts, be).reshape(epad)
    agg, dinv2 = _sc_conv(xw, src3, dst3, npad, nb)
    pa2, pb2 = _tc_node2(agg, xw, dinv2[0].reshape(npad // r, 1, r), b1p,
                         w2ab, npad, r)
    pred = _sc_tail(pa2.reshape(npad), pb2.reshape(npad), dinv2, src3, dst3,
                    cflat, npad, nb, epw)
    return pred[:e]


# speedup vs baseline: 52.1296x; 1.1825x over previous
import functools

import jax
import jax.numpy as jnp
from jax import lax
from jax.experimental import pallas as pl
from jax.experimental.pallas import tpu as pltpu
from jax.experimental.pallas import tpu_sc as plsc

NC = 2
NS = 16
NW = NC * NS
L = 16
HP = 32
B = 80

f32 = jnp.float32
i32 = jnp.int32


def _mesh():
    return plsc.VectorSubcoreMesh(
        core_axis_name="c", subcore_axis_name="s", num_cores=NC, num_subcores=NS
    )


def _round_up(a, m):
    return (a + m - 1) // m * m


def _rsqrt_newton(d):
    y = plsc.bitcast(jnp.int32(0x5F3759DF) - (plsc.bitcast(d, i32) >> 1), f32)
    for _ in range(3):
        y = y * (1.5 - 0.5 * d * y * y)
    return y


def _sc_conv(xw, src3, dst3, npad, nb):
    nps = npad // NS

    @functools.partial(
        pl.kernel,
        out_type=[
            jax.ShapeDtypeStruct((NC, npad, HP), f32),
            jax.ShapeDtypeStruct((NC, npad), f32),
        ],
        mesh=_mesh(),
        scratch_types=[
            pltpu.VMEM((nb, B), i32),
            pltpu.VMEM((nb, B), i32),
            pltpu.VMEM((NC, nb, B), i32),
            pltpu.VMEM((B,), f32),
            pltpu.VMEM((nps,), f32),
            pltpu.VMEM((nps, HP), f32),
            pltpu.VMEM((B, HP), f32),
            pltpu.VMEM((B, HP), f32),
            pltpu.VMEM((nps, HP), f32),
            pltpu.VMEM_SHARED((npad,), f32),
            pltpu.VMEM_SHARED((npad, HP), f32),
            pltpu.VMEM_SHARED((npad, HP), f32),
            pltpu.SemaphoreType.DMA,
            pltpu.SemaphoreType.DMA,
        ],
        compiler_params=pltpu.CompilerParams(
            use_tc_tiling_on_sc=False, needs_layout_passes=False),
    )
    def conv_kernel(xw_hbm, src_hbm, dst_hbm, agg_out, dinv_out,
                    srcv, dstv, dsth, ones_v, dvs, xwb, rows0, rows1, stage,
                    deg_sh, tab_sh, agg_sh, sem0, sem1):
        c = lax.axis_index("c")
        s = lax.axis_index("s")
        wid = c * NS + s
        nsl = pl.ds(s * nps, nps)
        pltpu.sync_copy(src_hbm.at[wid], srcv)
        pltpu.sync_copy(dst_hbm.at[wid], dstv)
        pltpu.sync_copy(dst_hbm.at[pl.ds(NC * s, NC)], dsth)
        one = jnp.ones((L,), f32)
        zero = jnp.zeros((L,), f32)

        def fill_ones(i, carry):
            ones_v[pl.ds(i * L, L)] = one
            return carry

        lax.fori_loop(0, B // L, fill_ones, 0)

        def zero_dvs(i, carry):
            dvs[pl.ds(i * L, L)] = zero
            return carry

        lax.fori_loop(0, nps // L, zero_dvs, 0)

        def zero_stage(rr, carry):
            stage[rr, pl.ds(0, L)] = zero
            stage[rr, pl.ds(L, L)] = zero
            return carry

        lax.fori_loop(0, nps, zero_stage, 0)
        pltpu.sync_copy(dvs, deg_sh.at[nsl])
        pltpu.sync_copy(stage, agg_sh.at[nsl])
        plsc.subcore_barrier()

        nbb = nb // 5
        for q in range(NC):
            def deg_issue(jj, carry, q=q):
                for t in range(5):
                    pltpu.async_copy(ones_v, deg_sh.at[dsth.at[q, jj * 5 + t]],
                                     sem0, add=True)
                return carry

            def deg_drain(jj, carry, q=q):
                for t in range(5):
                    pltpu.make_async_copy(
                        ones_v, deg_sh.at[dsth.at[q, 0]], sem0).wait()
                return carry

            deg_issue(0, 0)

            def deg_step(jj, carry, q=q):
                deg_issue(jj, carry)
                deg_drain(jj, carry)
                return carry

            lax.fori_loop(1, nbb, deg_step, 0)
            deg_drain(0, 0)

            def deg_tail(j, carry, q=q):
                pltpu.sync_copy(ones_v, deg_sh.at[dsth.at[q, j]], add=True)
                return carry

            lax.fori_loop(nbb * 5, nb, deg_tail, 0)
        plsc.subcore_barrier()

        pltpu.sync_copy(deg_sh.at[nsl], dvs)
        pltpu.sync_copy(xw_hbm.at[pl.ds(s * nps, nps)], xwb)

        def mk_dinv(i, carry):
            sl = pl.ds(i * L, L)
            dvs[sl] = _rsqrt_newton(dvs[sl] + 1.0)
            return carry

        lax.fori_loop(0, nps // L, mk_dinv, 0)

        def scale_row(rr, carry):
            dvec = plsc.load_gather(dvs, [jnp.full((L,), rr, i32)])
            xwb[rr, pl.ds(0, L)] = xwb[rr, pl.ds(0, L)] * dvec
            xwb[rr, pl.ds(L, L)] = xwb[rr, pl.ds(L, L)] * dvec
            return carry

        lax.fori_loop(0, nps, scale_row, 0)
        pltpu.sync_copy(xwb, tab_sh.at[nsl])
        pltpu.sync_copy(dvs, dinv_out.at[c, nsl])
        plsc.subcore_barrier()

        def gat(j, rows, sem):
            return pltpu.make_async_copy(tab_sh.at[srcv.at[j]], rows, sem)

        gat(0, rows0, sem0).start()

        def blk(jj, carry):
            j = jj * 2
            gat(j, rows0, sem0).wait()
            gat(j + 1, rows1, sem1).start()
            pltpu.sync_copy(rows0, agg_sh.at[dstv.at[j]], add=True)
            gat(j + 1, rows1, sem1).wait()
            gat(j + 2, rows0, sem0).start()
            pltpu.sync_copy(rows1, agg_sh.at[dstv.at[j + 1]], add=True)
            return carry

        npair = (nb - 1) // 2
        lax.fori_loop(0, npair, blk, 0)

        def tail(j, carry):
            gat(j, rows0, sem0).wait()
            pltpu.sync_copy(rows0, agg_sh.at[dstv.at[j]], add=True)
            gat(j + 1, rows0, sem0).start()
            return carry

        lax.fori_loop(2 * npair, nb - 1, tail, 0)
        gat(nb - 1, rows0, sem0).wait()
        pltpu.sync_copy(rows0, agg_sh.at[dstv.at[nb - 1]], add=True)
        plsc.subcore_barrier()
        pltpu.sync_copy(agg_sh.at[nsl], stage)
        pltpu.sync_copy(stage, agg_out.at[c, nsl])

    return conv_kernel(xw, src3, dst3)


def _sc_tail(pa, pb, dinv2, src3, dst3, cflat, npad, nb, epw):
    nps = npad // NS

    @functools.partial(
        pl.kernel,
        out_type=jax.ShapeDtypeStruct((NW * epw,), f32),
        mesh=_mesh(),
        scratch_types=[
            pltpu.VMEM((npad,), f32),
            pltpu.VMEM((npad,), f32),
            pltpu.VMEM((nps,), f32),
            pltpu.VMEM((nps,), f32),
            pltpu.VMEM((nb, B), i32),
            pltpu.VMEM((nb, B), i32),
            pltpu.VMEM((5 * B,), f32),
            pltpu.VMEM((5 * B,), f32),
            pltpu.VMEM((5 * B,), f32),
            pltpu.VMEM((5 * B,), f32),
            pltpu.SemaphoreType.DMA,
            pltpu.SemaphoreType.DMA,
            pltpu.VMEM((nps,), f32),
            pltpu.VMEM((nps,), f32),
            pltpu.VMEM((epw,), f32),
            pltpu.VMEM((epw,), f32),
            pltpu.VMEM_SHARED((npad,), f32),
            pltpu.VMEM_SHARED((npad,), f32),
        ],
        compiler_params=pltpu.CompilerParams(
            use_tc_tiling_on_sc=False, needs_layout_passes=False),
    )
    def tail_kernel(pa_hbm, pb_hbm, dinv_hbm, src_hbm, dst_hbm, c_hbm,
                    out_hbm, sa_t, sb_t, pav, pbv, srcq, dstq, va0, vb0,
                    va1, vb1, semA, semB, dv_s, ha_s, cg, ov, sh_a, sh_b):
        c = lax.axis_index("c")
        s = lax.axis_index("s")
        wid = c * NS + s
        nsl = pl.ds(s * nps, nps)
        pltpu.sync_copy(pa_hbm, sa_t)
        pltpu.sync_copy(pb_hbm, sb_t)
        pltpu.sync_copy(dinv_hbm.at[0, nsl], dv_s)
        pltpu.sync_copy(pa_hbm.at[nsl], pav)
        pltpu.sync_copy(pb_hbm.at[nsl], pbv)
        zero = jnp.zeros((L,), f32)

        def zero_ha(i, carry):
            ha_s[pl.ds(i * L, L)] = zero
            return carry

        lax.fori_loop(0, nps // L, zero_ha, 0)
        pltpu.sync_copy(ha_s, sh_a.at[nsl])
        pltpu.sync_copy(ha_s, sh_b.at[nsl])
        plsc.subcore_barrier()

        nbt = nb // 5

        def gather_batch(jb, va, vb):
            for t in range(5):
                for k in range(B // L):
                    vsl = pl.ds(t * B + k * L, L)
                    idx = srcq[5 * jb + t, pl.ds(k * L, L)]
                    va[vsl] = plsc.load_gather(sa_t, [idx])
                    vb[vsl] = plsc.load_gather(sb_t, [idx])

        def issue_batch(jb, va, vb, sem):
            for t in range(5):
                vsl = pl.ds(t * B, B)
                pltpu.async_copy(va.at[vsl], sh_a.at[dstq.at[5 * jb + t]],
                                 sem, add=True)
                pltpu.async_copy(vb.at[vsl], sh_b.at[dstq.at[5 * jb + t]],
                                 sem, add=True)

        def drain_batch(va, vb, sem):
            for t in range(5):
                vsl = pl.ds(t * B, B)
                pltpu.make_async_copy(va.at[vsl], sh_a.at[dstq.at[0]],
                                      sem).wait()
                pltpu.make_async_copy(vb.at[vsl], sh_b.at[dstq.at[0]],
                                      sem).wait()

        for q in range(NC):
            pltpu.sync_copy(src_hbm.at[NC * s + q], srcq)
            pltpu.sync_copy(dst_hbm.at[NC * s + q], dstq)
            gather_batch(0, va0, vb0)

            def hist_pair(jj, carry):
                jb = jj * 2
                issue_batch(jb, va0, vb0, semA)
                gather_batch(jb + 1, va1, vb1)
                issue_batch(jb + 1, va1, vb1, semB)
                drain_batch(va0, vb0, semA)
                gather_batch(jb + 2, va0, vb0)
                drain_batch(va1, vb1, semB)
                return carry

            npair_h = (nbt - 1) // 2
            lax.fori_loop(0, npair_h, hist_pair, 0)

            def hist_tail(jb, carry):
                gather_batch(jb, va0, vb0)
                issue_batch(jb, va0, vb0, semA)
                drain_batch(va0, vb0, semA)
                return carry

            lax.fori_loop(2 * npair_h, nbt, hist_tail, 0)

            def hist_rem(j, carry):
                for k in range(B // L):
                    ksl = pl.ds(k * L, L)
                    idx = srcq[j, ksl]
                    va0[ksl] = plsc.load_gather(sa_t, [idx])
                    vb0[ksl] = plsc.load_gather(sb_t, [idx])
                pltpu.sync_copy(va0.at[pl.ds(0, B)], sh_a.at[dstq.at[j]],
                                add=True)
                pltpu.sync_copy(vb0.at[pl.ds(0, B)], sh_b.at[dstq.at[j]],
                                add=True)
                return carry

            lax.fori_loop(nbt * 5, nb, hist_rem, 0)
        plsc.subcore_barrier()

        pltpu.sync_copy(sh_a.at[nsl], ha_s)

        def mk_sa(i, carry):
            sl = pl.ds(i * L, L)
            ha_s[sl] = dv_s[sl] * (ha_s[sl] + pav[sl])
            return carry

        lax.fori_loop(0, nps // L, mk_sa, 0)
        pltpu.sync_copy(ha_s, sh_a.at[nsl])
        pltpu.sync_copy(sh_b.at[nsl], ha_s)

        def mk_sb(i, carry):
            sl = pl.ds(i * L, L)
            ha_s[sl] = dv_s[sl] * (ha_s[sl] + pbv[sl])
            return carry

        lax.fori_loop(0, nps // L, mk_sb, 0)
        pltpu.sync_copy(ha_s, sh_b.at[nsl])
        plsc.subcore_barrier()
        pltpu.sync_copy(sh_a, sa_t)
        pltpu.sync_copy(sh_b, sb_t)

        pltpu.sync_copy(src_hbm.at[wid], srcq)
        pltpu.sync_copy(dst_hbm.at[wid], dstq)
        pltpu.sync_copy(c_hbm.at[pl.ds(wid * epw, epw)], cg)

        def gatf(j, carry):
            for k in range(B // L):
                ksl = pl.ds(k * L, L)
                sl = pl.ds(j * B + k * L, L)
                ov[sl] = (plsc.load_gather(sa_t, [srcq[j, ksl]])
                          + plsc.load_gather(sb_t, [dstq[j, ksl]]) + cg[sl])
            return carry

        lax.fori_loop(0, nb, gatf, 0)
        pltpu.sync_copy(ov, out_hbm.at[pl.ds(wid * epw, epw)])

    return tail_kernel(pa, pb, dinv2, src3, dst3, cflat)


def _tc_edge(eaT, We1pT, be1c, wcc, consts, be):
    de, epad = eaT.shape
    nblk = epad // be

    def body(ea_ref, w_ref, b_ref, wc_ref, k_ref, out_ref):
        t = jnp.dot(w_ref[...], ea_ref[...], preferred_element_type=f32,
                    precision=jax.lax.Precision.HIGHEST)
        t = jnp.maximum(t + b_ref[...], 0.0)
        out_ref[0, 0, :] = jnp.sum(t * wc_ref[...], axis=0) + k_ref[0, 0]

    return pl.pallas_call(
        body,
        grid=(nblk,),
        in_specs=[
            pl.BlockSpec((de, be), lambda i: (0, i)),
            pl.BlockSpec((HP, de), lambda i: (0, 0)),
            pl.BlockSpec((HP, 1), lambda i: (0, 0)),
            pl.BlockSpec((HP, 1), lambda i: (0, 0)),
            pl.BlockSpec(memory_space=pltpu.SMEM),
        ],
        out_specs=pl.BlockSpec((1, 1, be), lambda i: (i, 0, 0)),
        out_shape=jax.ShapeDtypeStruct((nblk, 1, be), f32),
    )(eaT, We1pT, be1c, wcc, consts)


def _tc_xw(xp, W1p, npad, r):
    nblk = npad // r
    d = xp.shape[1]

    def body(x_ref, w_ref, xw_ref):
        xw_ref[...] = jnp.dot(x_ref[...], w_ref[...],
                              preferred_element_type=f32)

    return pl.pallas_call(
        body,
        grid=(nblk,),
        in_specs=[
            pl.BlockSpec((r, d), lambda i: (i, 0)),
            pl.BlockSpec((d, HP), lambda i: (0, 0)),
        ],
        out_specs=pl.BlockSpec((r, HP), lambda i: (i, 0)),
        out_shape=jax.ShapeDtypeStruct((npad, HP), f32),
    )(xp, W1p)


def _tc_node2(agg, xw, dinv3, b1p, w2ab, npad, r):
    nblk = npad // r

    def body(agg_ref, xw_ref, dinv_ref, b_ref, w_ref, pa_ref, pb_ref):
        dinv = dinv_ref[0, 0, :]
        xs = xw_ref[...] * dinv[:, None]
        pre = (agg_ref[0] + agg_ref[1] + xs) * dinv[:, None]
        h1 = jnp.maximum(pre + b_ref[...], 0.0)
        pa_ref[0, 0, :] = dinv * jnp.sum(h1 * w_ref[0:1, :], axis=1)
        pb_ref[0, 0, :] = dinv * jnp.sum(h1 * w_ref[1:2, :], axis=1)

    return pl.pallas_call(
        body,
        grid=(nblk,),
        in_specs=[
            pl.BlockSpec((2, r, HP), lambda i: (0, i, 0)),
            pl.BlockSpec((r, HP), lambda i: (i, 0)),
            pl.BlockSpec((1, 1, r), lambda i: (i, 0, 0)),
            pl.BlockSpec((1, HP), lambda i: (0, 0)),
            pl.BlockSpec((2, HP), lambda i: (0, 0)),
        ],
        out_specs=[
            pl.BlockSpec((1, 1, r), lambda i: (i, 0, 0)),
            pl.BlockSpec((1, 1, r), lambda i: (i, 0, 0)),
        ],
        out_shape=[
            jax.ShapeDtypeStruct((nblk, 1, r), f32),
            jax.ShapeDtypeStruct((nblk, 1, r), f32),
        ],
    )(agg, xw, dinv3, b1p, w2ab)


def kernel(x, edge_index, edge_attr, W1, b1, W2, b2, We1, be1, We2, be2, Wf, bf):
    n, d = x.shape
    e = edge_index.shape[1]
    h = W1.shape[1]

    r = 1024
    npad = _round_up(n + 1, r)
    nb = _round_up(e, NW * B) // (NW * B)
    epw = nb * B
    epad = NW * epw
    be = max(NW * B, 2560)
    while epad % be:
        be //= 2

    pad_h = HP - h
    W1p = jnp.pad(W1, ((0, 0), (0, pad_h)))
    We1pT = jnp.pad(We1, ((0, 0), (0, pad_h))).T
    be1c = jnp.pad(be1, (0, pad_h)).reshape(HP, 1)
    Wfa, Wfb, Wfc = Wf[:h, 0], Wf[h:2 * h, 0], Wf[2 * h:, 0]
    b1p = jnp.pad(b1, (0, pad_h)).reshape(1, HP)
    w2ab = jnp.pad(jnp.stack([W2 @ Wfa, W2 @ Wfb]), ((0, 0), (0, pad_h)))
    wcc = jnp.pad(We2 @ Wfc, (0, pad_h)).reshape(HP, 1)
    consts = (be2 @ Wfc + bf[0] + b2 @ Wfa + b2 @ Wfb).reshape(1, 1)

    xp = jnp.pad(x, ((0, npad - n), (0, 0)))
    eaT = jnp.pad(edge_attr, ((0, epad - e), (0, 0))).T
    src = jnp.pad(edge_index[0], (0, epad - e), constant_values=n)
    dst = jnp.pad(edge_index[1], (0, epad - e), constant_values=n)
    src3 = src.reshape(NW, nb, B)
    dst3 = dst.reshape(NW, nb, B)

    xw = _tc_xw(xp, W1p, npad, r)
    cflat = _tc_edge(eaT, We1pT, be1c, wcc, consts, be).reshape(epad)
    agg, dinv2 = _sc_conv(xw, src3, dst3, npad, nb)
    pa2, pb2 = _tc_node2(agg, xw, dinv2[0].reshape(npad // r, 1, r), b1p,
                         w2ab, npad, r)
    pred = _sc_tail(pa2.reshape(npad), pb2.reshape(npad), dinv2, src3, dst3,
                    cflat, npad, nb, epw)
    return pred[:e]
